# baseline (speedup 1.0000x reference)
import functools

import jax
import jax.numpy as jnp
from jax import lax
from jax.experimental import pallas as pl
from jax.experimental.pallas import tpu as pltpu
from jax.experimental.pallas import tpu_sc as plsc

Q = 1024
K = 100000
D = 256
C = 100
BK = 2000
NBLK = K // BK
THRESH = 20.0

FLAT_ROWS = K * C // 128

def _dist_argmin_body(x_ref, keys_ref, xsq_ref, ksq_ref, w_ref, b_ref,
                      dist_ref, idx_ref, mlog_ref, idx2_ref):
    pid = pl.program_id(0)

    @pl.when(pid == 0)
    def _init():
        mlog_ref[...] = (
            lax.dot_general(x_ref[...], w_ref[...],
                            (((1,), (0,)), ((), ())))
            + b_ref[...]
        )

    x = x_ref[...]
    keys = keys_ref[...]
    s2 = lax.dot_general(x * -2.0, keys, (((1,), (1,)), ((), ())))
    d2 = (xsq_ref[...] + s2) + ksq_ref[0]
    dist = jnp.sqrt(d2)

    m = jnp.min(dist, axis=1, keepdims=True)
    cols = lax.broadcasted_iota(
        jnp.int32, (1, BK), 1).astype(jnp.float32)
    bidx = jnp.min(jnp.where(dist == m, cols, jnp.float32(BK)),
                   axis=1, keepdims=True).astype(jnp.int32)
    gidx = bidx + pid * BK

    @pl.when(pid == 0)
    def _first():
        dist_ref[...] = m
        idx_ref[...] = gidx

    @pl.when(pid > 0)
    def _update():
        better = m < dist_ref[...]
        dist_ref[...] = jnp.where(better, m, dist_ref[...])
        idx_ref[...] = jnp.where(better, gidx, idx_ref[...])

    @pl.when(pid == NBLK - 1)
    def _emit_rows():
        flat = idx_ref[...] * C
        r0 = flat // 128
        r1 = jnp.minimum(r0 + 1, FLAT_ROWS - 1)
        idx2_ref[...] = jnp.concatenate([r0, r1], axis=1)


def _dist_argmin(x, cache_keys, W, b2d):
    xsq = jnp.sum(x * x, axis=1, keepdims=True)
    ksq = jnp.sum(cache_keys * cache_keys, axis=1).reshape(NBLK, 1, BK)
    return pl.pallas_call(
        _dist_argmin_body,
        grid=(NBLK,),
        in_specs=[
            pl.BlockSpec((Q, D), lambda i: (0, 0)),
            pl.BlockSpec((BK, D), lambda i: (i, 0)),
            pl.BlockSpec((Q, 1), lambda i: (0, 0)),
            pl.BlockSpec((1, 1, BK), lambda i: (i, 0, 0)),
            pl.BlockSpec((D, C), lambda i: (0, 0)),
            pl.BlockSpec((1, C), lambda i: (0, 0)),
        ],
        out_specs=[
            pl.BlockSpec((Q, 1), lambda i: (0, 0)),
            pl.BlockSpec((Q, 1), lambda i: (0, 0)),
            pl.BlockSpec((Q, C), lambda i: (0, 0)),
            pl.BlockSpec((Q, 2), lambda i: (0, 0)),
        ],
        out_shape=[
            jax.ShapeDtypeStruct((Q, 1), jnp.float32),
            jax.ShapeDtypeStruct((Q, 1), jnp.int32),
            jax.ShapeDtypeStruct((Q, C), jnp.float32),
            jax.ShapeDtypeStruct((Q, 2), jnp.int32),
        ],
    )(x, cache_keys, xsq, ksq, W, b2d)


_NW = 32
_NIDX = 2 * Q
_BPW = _NIDX // _NW


def _gather_rows(table_flat, idx2_flat):
    mesh = plsc.VectorSubcoreMesh(core_axis_name="c", subcore_axis_name="s")

    @functools.partial(
        pl.kernel,
        mesh=mesh,
        out_type=jax.ShapeDtypeStruct((_NIDX, 128), jnp.float32),
        scratch_types=[
            pltpu.VMEM((_BPW,), jnp.int32),
            pltpu.VMEM((_BPW, 128), jnp.float32),
            pltpu.SemaphoreType.DMA,
        ],
    )
    def _k(table_hbm, idx_hbm, out_hbm, idx_v, rows_v, sem):
        wid = lax.axis_index("s") * 2 + lax.axis_index("c")
        base = wid * _BPW
        pltpu.sync_copy(idx_hbm.at[pl.ds(base, _BPW)], idx_v)
        pltpu.async_copy(table_hbm.at[idx_v], rows_v, sem).wait()
        pltpu.sync_copy(rows_v, out_hbm.at[pl.ds(base, _BPW)])

    return _k(table_flat, idx2_flat)


def _finalize_body(mlog_ref, buf_ref, dist_ref, idx_ref,
                   probs_ref, cache_ref):
    off = (idx_ref[...] * C) % 128
    buf = buf_ref[...]
    clog = buf[:, 0:C]
    for s in range(1, 32):
        clog = jnp.where(off == 4 * s, buf[:, 4 * s:4 * s + C], clog)

    is_cache = dist_ref[...] <= THRESH
    logits = jnp.where(is_cache, clog, mlog_ref[...])
    m = jnp.max(logits, axis=1, keepdims=True)
    e = jnp.exp(logits - m)
    probs_ref[...] = e / jnp.sum(e, axis=1, keepdims=True)
    cache_ref[...] = is_cache


def _finalize(mlog, buf, dist, idx):
    return pl.pallas_call(
        _finalize_body,
        out_shape=[
            jax.ShapeDtypeStruct((Q, C), jnp.float32),
            jax.ShapeDtypeStruct((Q, 1), jnp.bool_),
        ],
    )(mlog, buf, dist, idx)


def kernel(x, cache_keys, cache_preds, W, b):
    dist, idx, mlog, idx2 = _dist_argmin(x, cache_keys, W, b.reshape(1, C))
    rows = _gather_rows(cache_preds.reshape(FLAT_ROWS, 128),
                        idx2.reshape(_NIDX))
    probs, is_cache = _finalize(mlog, rows.reshape(Q, 256), dist, idx)
    return probs, is_cache.reshape(Q)


# speedup vs baseline: 1.1797x; 1.1797x over previous
import functools

import jax
import jax.numpy as jnp
from jax import lax
from jax.experimental import pallas as pl
from jax.experimental.pallas import tpu as pltpu
from jax.experimental.pallas import tpu_sc as plsc

Q = 1024
K = 100000
D = 256
C = 100
BK = 2000
NBLK = K // BK
THRESH = 20.0

def _dist_argmin_body(x_ref, keys_ref, xsq_ref, ksq_ref, w_ref, b_ref,
                      dist_ref, idx_ref, mlog_ref):
    pid = pl.program_id(0)

    @pl.when(pid == 0)
    def _init():
        mlog_ref[...] = (
            lax.dot_general(x_ref[...], w_ref[...],
                            (((1,), (0,)), ((), ())))
            + b_ref[...]
        )

    x = x_ref[...]
    keys = keys_ref[...]
    s2 = lax.dot_general(x * -2.0, keys, (((1,), (1,)), ((), ())))
    d2 = (xsq_ref[...] + s2) + ksq_ref[0]
    dist = jnp.sqrt(d2)

    m = jnp.min(dist, axis=1, keepdims=True)
    cols = lax.broadcasted_iota(
        jnp.int32, (1, BK), 1).astype(jnp.float32)
    bidx = jnp.min(jnp.where(dist == m, cols, jnp.float32(BK)),
                   axis=1, keepdims=True).astype(jnp.int32)
    gidx = bidx + pid * BK

    @pl.when(pid == 0)
    def _first():
        dist_ref[...] = m
        idx_ref[...] = gidx

    @pl.when(pid > 0)
    def _update():
        better = m < dist_ref[...]
        dist_ref[...] = jnp.where(better, m, dist_ref[...])
        idx_ref[...] = jnp.where(better, gidx, idx_ref[...])


def _dist_argmin(x, cache_keys, W, b2d):
    xsq = jnp.sum(x * x, axis=1, keepdims=True)
    ksq = jnp.sum(cache_keys * cache_keys, axis=1).reshape(NBLK, 1, BK)
    return pl.pallas_call(
        _dist_argmin_body,
        grid=(NBLK,),
        in_specs=[
            pl.BlockSpec((Q, D), lambda i: (0, 0)),
            pl.BlockSpec((BK, D), lambda i: (i, 0)),
            pl.BlockSpec((Q, 1), lambda i: (0, 0)),
            pl.BlockSpec((1, 1, BK), lambda i: (i, 0, 0)),
            pl.BlockSpec((D, C), lambda i: (0, 0)),
            pl.BlockSpec((1, C), lambda i: (0, 0)),
        ],
        out_specs=[
            pl.BlockSpec((Q, 1), lambda i: (0, 0)),
            pl.BlockSpec((Q, 1), lambda i: (0, 0)),
            pl.BlockSpec((Q, C), lambda i: (0, 0)),
        ],
        out_shape=[
            jax.ShapeDtypeStruct((Q, 1), jnp.float32),
            jax.ShapeDtypeStruct((Q, 1), jnp.int32),
            jax.ShapeDtypeStruct((Q, C), jnp.float32),
        ],
    )(x, cache_keys, xsq, ksq, W, b2d)


_NW = 32
_BPW = Q // _NW


def _gather_rows(table, idx):
    mesh = plsc.VectorSubcoreMesh(core_axis_name="c", subcore_axis_name="s")

    @functools.partial(
        pl.kernel,
        mesh=mesh,
        out_type=jax.ShapeDtypeStruct((Q, 128), jnp.float32),
        scratch_types=[
            pltpu.VMEM((_BPW,), jnp.int32),
            pltpu.VMEM((_BPW, 128), jnp.float32),
            pltpu.SemaphoreType.DMA,
        ],
    )
    def _k(table_hbm, idx_hbm, out_hbm, idx_v, rows_v, sem):
        wid = lax.axis_index("s") * 2 + lax.axis_index("c")
        base = wid * _BPW
        pltpu.sync_copy(idx_hbm.at[pl.ds(base, _BPW)], idx_v)
        pltpu.async_copy(table_hbm.at[idx_v], rows_v, sem).wait()
        pltpu.sync_copy(rows_v, out_hbm.at[pl.ds(base, _BPW)])

    return _k(table, idx)


def _finalize_body(mlog_ref, buf_ref, dist_ref, probs_ref, cache_ref):
    clog = buf_ref[...][:, :C]
    is_cache = dist_ref[...] <= THRESH
    logits = jnp.where(is_cache, clog, mlog_ref[...])
    m = jnp.max(logits, axis=1, keepdims=True)
    e = jnp.exp(logits - m)
    probs_ref[...] = e / jnp.sum(e, axis=1, keepdims=True)
    cache_ref[...] = is_cache


def _finalize(mlog, buf, dist):
    return pl.pallas_call(
        _finalize_body,
        out_shape=[
            jax.ShapeDtypeStruct((Q, C), jnp.float32),
            jax.ShapeDtypeStruct((Q, 1), jnp.bool_),
        ],
    )(mlog, buf, dist)


def kernel(x, cache_keys, cache_preds, W, b):
    dist, idx, mlog = _dist_argmin(x, cache_keys, W, b.reshape(1, C))
    table = jnp.pad(cache_preds, ((0, 0), (0, 128 - C)))
    rows = _gather_rows(table, idx.reshape(Q))
    probs, is_cache = _finalize(mlog, rows, dist)
    return probs, is_cache.reshape(Q)


# speedup vs baseline: 1.4894x; 1.2625x over previous
import functools

import jax
import jax.numpy as jnp
from jax import lax
from jax.experimental import pallas as pl
from jax.experimental.pallas import tpu as pltpu
from jax.experimental.pallas import tpu_sc as plsc

Q = 1024
K = 100000
D = 256
C = 100
BK = 2000
NBLK = K // BK
THRESH = 20.0

def _dist_argmin_body(x_ref, keys_ref, xsq_ref, ksq_ref, w_ref, b_ref,
                      dist_ref, idx_ref, mlog_ref):
    pid = pl.program_id(0)

    @pl.when(pid == 0)
    def _init():
        mlog_ref[...] = (
            lax.dot_general(x_ref[...], w_ref[...],
                            (((1,), (0,)), ((), ())))
            + b_ref[...]
        )

    x = x_ref[...]
    keys = keys_ref[...]
    s2 = lax.dot_general(x * -2.0, keys, (((1,), (1,)), ((), ())))
    d2 = (xsq_ref[...] + s2) + ksq_ref[0]

    m_d2 = jnp.min(d2, axis=1, keepdims=True)
    m = jnp.sqrt(m_d2)
    mb = lax.bitcast_convert_type(m_d2, jnp.int32)
    c1 = lax.bitcast_convert_type(mb + 1, jnp.float32)
    c2 = lax.bitcast_convert_type(mb + 2, jnp.float32)
    B = jnp.where(jnp.sqrt(c1) == m,
                  jnp.where(jnp.sqrt(c2) == m, c2, c1), m_d2)
    cols = lax.broadcasted_iota(
        jnp.int32, (1, BK), 1).astype(jnp.float32)
    bidx = jnp.min(jnp.where(d2 <= B, cols, jnp.float32(BK)),
                   axis=1, keepdims=True).astype(jnp.int32)
    gidx = bidx + pid * BK

    @pl.when(pid == 0)
    def _first():
        dist_ref[...] = m
        idx_ref[...] = gidx

    @pl.when(pid > 0)
    def _update():
        better = m < dist_ref[...]
        dist_ref[...] = jnp.where(better, m, dist_ref[...])
        idx_ref[...] = jnp.where(better, gidx, idx_ref[...])


def _dist_argmin(x, cache_keys, W, b2d):
    xsq = jnp.sum(x * x, axis=1, keepdims=True)
    ksq = jnp.sum(cache_keys * cache_keys, axis=1).reshape(NBLK, 1, BK)
    return pl.pallas_call(
        _dist_argmin_body,
        grid=(NBLK,),
        in_specs=[
            pl.BlockSpec((Q, D), lambda i: (0, 0)),
            pl.BlockSpec((BK, D), lambda i: (i, 0)),
            pl.BlockSpec((Q, 1), lambda i: (0, 0)),
            pl.BlockSpec((1, 1, BK), lambda i: (i, 0, 0)),
            pl.BlockSpec((D, C), lambda i: (0, 0)),
            pl.BlockSpec((1, C), lambda i: (0, 0)),
        ],
        out_specs=[
            pl.BlockSpec((Q, 1), lambda i: (0, 0)),
            pl.BlockSpec((Q, 1), lambda i: (0, 0)),
            pl.BlockSpec((Q, C), lambda i: (0, 0)),
        ],
        out_shape=[
            jax.ShapeDtypeStruct((Q, 1), jnp.float32),
            jax.ShapeDtypeStruct((Q, 1), jnp.int32),
            jax.ShapeDtypeStruct((Q, C), jnp.float32),
        ],
    )(x, cache_keys, xsq, ksq, W, b2d)


_NW = 32
_BPW = Q // _NW


def _gather_rows(table, idx):
    mesh = plsc.VectorSubcoreMesh(core_axis_name="c", subcore_axis_name="s")

    @functools.partial(
        pl.kernel,
        mesh=mesh,
        out_type=jax.ShapeDtypeStruct((Q, 128), jnp.float32),
        scratch_types=[
            pltpu.VMEM((_BPW,), jnp.int32),
            pltpu.VMEM((_BPW, 128), jnp.float32),
            pltpu.SemaphoreType.DMA,
        ],
    )
    def _k(table_hbm, idx_hbm, out_hbm, idx_v, rows_v, sem):
        wid = lax.axis_index("s") * 2 + lax.axis_index("c")
        base = wid * _BPW
        pltpu.sync_copy(idx_hbm.at[pl.ds(base, _BPW)], idx_v)
        pltpu.async_copy(table_hbm.at[idx_v], rows_v, sem).wait()
        pltpu.sync_copy(rows_v, out_hbm.at[pl.ds(base, _BPW)])

    return _k(table, idx)


def _finalize_body(mlog_ref, buf_ref, dist_ref, probs_ref, cache_ref):
    clog = buf_ref[...][:, :C]
    is_cache = dist_ref[...] <= THRESH
    logits = jnp.where(is_cache, clog, mlog_ref[...])
    m = jnp.max(logits, axis=1, keepdims=True)
    e = jnp.exp(logits - m)
    probs_ref[...] = e / jnp.sum(e, axis=1, keepdims=True)
    cache_ref[...] = is_cache


def _finalize(mlog, buf, dist):
    return pl.pallas_call(
        _finalize_body,
        out_shape=[
            jax.ShapeDtypeStruct((Q, C), jnp.float32),
            jax.ShapeDtypeStruct((Q, 1), jnp.bool_),
        ],
    )(mlog, buf, dist)


def kernel(x, cache_keys, cache_preds, W, b):
    dist, idx, mlog = _dist_argmin(x, cache_keys, W, b.reshape(1, C))
    table = jnp.pad(cache_preds, ((0, 0), (0, 128 - C)))
    rows = _gather_rows(table, idx.reshape(Q))
    probs, is_cache = _finalize(mlog, rows, dist)
    return probs, is_cache.reshape(Q)


# speedup vs baseline: 1.6388x; 1.1003x over previous
import functools

import jax
import jax.numpy as jnp
from jax import lax
from jax.experimental import pallas as pl
from jax.experimental.pallas import tpu as pltpu
from jax.experimental.pallas import tpu_sc as plsc

Q = 1024
K = 100000
D = 256
C = 100
BK = 4000
NBLK = K // BK
THRESH = 20.0

def _dist_argmin_body(x_ref, keys_ref, xsq_ref, ksq_ref, w_ref, b_ref,
                      dist_ref, idx_ref, mlog_ref):
    pid = pl.program_id(0)

    @pl.when(pid == 0)
    def _init():
        mlog_ref[...] = (
            lax.dot_general(x_ref[...], w_ref[...],
                            (((1,), (0,)), ((), ())))
            + b_ref[...]
        )

    x = x_ref[...]
    keys = keys_ref[...]
    s2 = lax.dot_general(x * -2.0, keys, (((1,), (1,)), ((), ())))
    d2 = (xsq_ref[...] + s2) + ksq_ref[0]

    m_d2 = jnp.min(d2, axis=1, keepdims=True)
    m = jnp.sqrt(m_d2)
    mb = lax.bitcast_convert_type(m_d2, jnp.int32)
    c1 = lax.bitcast_convert_type(mb + 1, jnp.float32)
    c2 = lax.bitcast_convert_type(mb + 2, jnp.float32)
    B = jnp.where(jnp.sqrt(c1) == m,
                  jnp.where(jnp.sqrt(c2) == m, c2, c1), m_d2)
    cols = lax.broadcasted_iota(
        jnp.int32, (1, BK), 1).astype(jnp.float32)
    bidx = jnp.min(jnp.where(d2 <= B, cols, jnp.float32(BK)),
                   axis=1, keepdims=True).astype(jnp.int32)
    gidx = bidx + pid * BK

    @pl.when(pid == 0)
    def _first():
        dist_ref[...] = m
        idx_ref[...] = gidx

    @pl.when(pid > 0)
    def _update():
        better = m < dist_ref[...]
        dist_ref[...] = jnp.where(better, m, dist_ref[...])
        idx_ref[...] = jnp.where(better, gidx, idx_ref[...])


def _dist_argmin(x, cache_keys, W, b2d):
    xsq = jnp.sum(x * x, axis=1, keepdims=True)
    ksq = jnp.sum(cache_keys * cache_keys, axis=1).reshape(NBLK, 1, BK)
    return pl.pallas_call(
        _dist_argmin_body,
        grid=(NBLK,),
        in_specs=[
            pl.BlockSpec((Q, D), lambda i: (0, 0)),
            pl.BlockSpec((BK, D), lambda i: (i, 0)),
            pl.BlockSpec((Q, 1), lambda i: (0, 0)),
            pl.BlockSpec((1, 1, BK), lambda i: (i, 0, 0)),
            pl.BlockSpec((D, C), lambda i: (0, 0)),
            pl.BlockSpec((1, C), lambda i: (0, 0)),
        ],
        out_specs=[
            pl.BlockSpec((Q, 1), lambda i: (0, 0)),
            pl.BlockSpec((Q, 1), lambda i: (0, 0)),
            pl.BlockSpec((Q, C), lambda i: (0, 0)),
        ],
        out_shape=[
            jax.ShapeDtypeStruct((Q, 1), jnp.float32),
            jax.ShapeDtypeStruct((Q, 1), jnp.int32),
            jax.ShapeDtypeStruct((Q, C), jnp.float32),
        ],
    )(x, cache_keys, xsq, ksq, W, b2d)


_NW = 32
_BPW = Q // _NW


def _gather_rows(table, idx):
    mesh = plsc.VectorSubcoreMesh(core_axis_name="c", subcore_axis_name="s")

    @functools.partial(
        pl.kernel,
        mesh=mesh,
        out_type=jax.ShapeDtypeStruct((Q, 128), jnp.float32),
        scratch_types=[
            pltpu.VMEM((_BPW,), jnp.int32),
            pltpu.VMEM((_BPW, 128), jnp.float32),
            pltpu.SemaphoreType.DMA,
        ],
    )
    def _k(table_hbm, idx_hbm, out_hbm, idx_v, rows_v, sem):
        wid = lax.axis_index("s") * 2 + lax.axis_index("c")
        base = wid * _BPW
        pltpu.sync_copy(idx_hbm.at[pl.ds(base, _BPW)], idx_v)
        pltpu.async_copy(table_hbm.at[idx_v], rows_v, sem).wait()
        pltpu.sync_copy(rows_v, out_hbm.at[pl.ds(base, _BPW)])

    return _k(table, idx)


def _finalize_body(mlog_ref, buf_ref, dist_ref, probs_ref, cache_ref):
    clog = buf_ref[...][:, :C]
    is_cache = dist_ref[...] <= THRESH
    logits = jnp.where(is_cache, clog, mlog_ref[...])
    m = jnp.max(logits, axis=1, keepdims=True)
    e = jnp.exp(logits - m)
    probs_ref[...] = e / jnp.sum(e, axis=1, keepdims=True)
    cache_ref[...] = is_cache


def _finalize(mlog, buf, dist):
    return pl.pallas_call(
        _finalize_body,
        out_shape=[
            jax.ShapeDtypeStruct((Q, C), jnp.float32),
            jax.ShapeDtypeStruct((Q, 1), jnp.bool_),
        ],
    )(mlog, buf, dist)


def kernel(x, cache_keys, cache_preds, W, b):
    dist, idx, mlog = _dist_argmin(x, cache_keys, W, b.reshape(1, C))
    table = jnp.pad(cache_preds, ((0, 0), (0, 128 - C)))
    rows = _gather_rows(table, idx.reshape(Q))
    probs, is_cache = _finalize(mlog, rows, dist)
    return probs, is_cache.reshape(Q)


# speedup vs baseline: 1.6665x; 1.0169x over previous
import functools

import jax
import jax.numpy as jnp
from jax import lax
from jax.experimental import pallas as pl
from jax.experimental.pallas import tpu as pltpu
from jax.experimental.pallas import tpu_sc as plsc

Q = 1024
K = 100000
D = 256
C = 100
BK = 5000
NBLK = K // BK
THRESH = 20.0

def _dist_argmin_body(x_ref, keys_ref, xsq_ref, ksq_ref, w_ref, b_ref,
                      dist_ref, idx_ref, mlog_ref):
    pid = pl.program_id(0)

    @pl.when(pid == 0)
    def _init():
        mlog_ref[...] = (
            lax.dot_general(x_ref[...], w_ref[...],
                            (((1,), (0,)), ((), ())))
            + b_ref[...]
        )

    x = x_ref[...]
    keys = keys_ref[...]
    s2 = lax.dot_general(x * -2.0, keys, (((1,), (1,)), ((), ())))
    d2 = (xsq_ref[...] + s2) + ksq_ref[0]

    m_d2 = jnp.min(d2, axis=1, keepdims=True)
    m = jnp.sqrt(m_d2)
    mb = lax.bitcast_convert_type(m_d2, jnp.int32)
    c1 = lax.bitcast_convert_type(mb + 1, jnp.float32)
    c2 = lax.bitcast_convert_type(mb + 2, jnp.float32)
    B = jnp.where(jnp.sqrt(c1) == m,
                  jnp.where(jnp.sqrt(c2) == m, c2, c1), m_d2)
    cols = lax.broadcasted_iota(
        jnp.int32, (1, BK), 1).astype(jnp.float32)
    bidx = jnp.min(jnp.where(d2 <= B, cols, jnp.float32(BK)),
                   axis=1, keepdims=True).astype(jnp.int32)
    gidx = bidx + pid * BK

    @pl.when(pid == 0)
    def _first():
        dist_ref[...] = m
        idx_ref[...] = gidx

    @pl.when(pid > 0)
    def _update():
        better = m < dist_ref[...]
        dist_ref[...] = jnp.where(better, m, dist_ref[...])
        idx_ref[...] = jnp.where(better, gidx, idx_ref[...])


def _dist_argmin(x, cache_keys, W, b2d):
    xsq = jnp.sum(x * x, axis=1, keepdims=True)
    ksq = jnp.sum(cache_keys * cache_keys, axis=1).reshape(NBLK, 1, BK)
    return pl.pallas_call(
        _dist_argmin_body,
        grid=(NBLK,),
        in_specs=[
            pl.BlockSpec((Q, D), lambda i: (0, 0)),
            pl.BlockSpec((BK, D), lambda i: (i, 0)),
            pl.BlockSpec((Q, 1), lambda i: (0, 0)),
            pl.BlockSpec((1, 1, BK), lambda i: (i, 0, 0)),
            pl.BlockSpec((D, C), lambda i: (0, 0)),
            pl.BlockSpec((1, C), lambda i: (0, 0)),
        ],
        out_specs=[
            pl.BlockSpec((Q, 1), lambda i: (0, 0)),
            pl.BlockSpec((Q, 1), lambda i: (0, 0)),
            pl.BlockSpec((Q, C), lambda i: (0, 0)),
        ],
        out_shape=[
            jax.ShapeDtypeStruct((Q, 1), jnp.float32),
            jax.ShapeDtypeStruct((Q, 1), jnp.int32),
            jax.ShapeDtypeStruct((Q, C), jnp.float32),
        ],
    )(x, cache_keys, xsq, ksq, W, b2d)


_NW = 32
_BPW = Q // _NW


def _gather_rows(table, idx):
    mesh = plsc.VectorSubcoreMesh(core_axis_name="c", subcore_axis_name="s")

    @functools.partial(
        pl.kernel,
        mesh=mesh,
        out_type=jax.ShapeDtypeStruct((Q, 128), jnp.float32),
        scratch_types=[
            pltpu.VMEM((_BPW,), jnp.int32),
            pltpu.VMEM((_BPW, 128), jnp.float32),
            pltpu.SemaphoreType.DMA,
        ],
    )
    def _k(table_hbm, idx_hbm, out_hbm, idx_v, rows_v, sem):
        wid = lax.axis_index("s") * 2 + lax.axis_index("c")
        base = wid * _BPW
        pltpu.sync_copy(idx_hbm.at[pl.ds(base, _BPW)], idx_v)
        pltpu.async_copy(table_hbm.at[idx_v], rows_v, sem).wait()
        pltpu.sync_copy(rows_v, out_hbm.at[pl.ds(base, _BPW)])

    return _k(table, idx)


def _finalize_body(mlog_ref, buf_ref, dist_ref, probs_ref, cache_ref):
    clog = buf_ref[...][:, :C]
    is_cache = dist_ref[...] <= THRESH
    logits = jnp.where(is_cache, clog, mlog_ref[...])
    m = jnp.max(logits, axis=1, keepdims=True)
    e = jnp.exp(logits - m)
    probs_ref[...] = e / jnp.sum(e, axis=1, keepdims=True)
    cache_ref[...] = is_cache


def _finalize(mlog, buf, dist):
    return pl.pallas_call(
        _finalize_body,
        out_shape=[
            jax.ShapeDtypeStruct((Q, C), jnp.float32),
            jax.ShapeDtypeStruct((Q, 1), jnp.bool_),
        ],
    )(mlog, buf, dist)


def kernel(x, cache_keys, cache_preds, W, b):
    dist, idx, mlog = _dist_argmin(x, cache_keys, W, b.reshape(1, C))
    table = jnp.pad(cache_preds, ((0, 0), (0, 128 - C)))
    rows = _gather_rows(table, idx.reshape(Q))
    probs, is_cache = _finalize(mlog, rows, dist)
    return probs, is_cache.reshape(Q)
